# Initial kernel scaffold; baseline (speedup 1.0000x reference)
import jax
import jax.numpy as jnp
import numpy as np
from jax.experimental import pallas as pl
from jax.experimental.pallas import tpu as pltpu

B = 2
S = 32
K = 3


def _matmul_body(g_ref, w_ref, b_ref, out_ref):
    k = pl.program_id(1)

    @pl.when(k == 0)
    def _init():
        out_ref[...] = b_ref[...] + jnp.dot(
            g_ref[0], w_ref[0], preferred_element_type=jnp.float32
        )

    @pl.when(k != 0)
    def _acc():
        out_ref[...] += jnp.dot(
            g_ref[0], w_ref[0], preferred_element_type=jnp.float32
        )


def kernel(feats, coords, W, b, num_frames):
    N, C = feats.shape
    T = num_frames
    NK = K * K * K * K
    coords32 = coords.astype(jnp.int32)
    first = coords32[:, 0]
    z, y, x = coords32[:, 1], coords32[:, 2], coords32[:, 3]
    frame = first % T

    lin = ((first * S + z) * S + y) * S + x
    grid = jnp.full((B * T * S * S * S,), -1, dtype=jnp.int32)
    grid = grid.at[lin].set(jnp.arange(N, dtype=jnp.int32))

    r = K // 2
    offs = []
    for kt in range(K):
        for kz in range(K):
            for ky in range(K):
                for kx in range(K):
                    offs.append((kt - r, kz - r, ky - r, kx - r))
    offs = np.array(offs, dtype=np.int32)

    dt = offs[:, 0][:, None]
    dz = offs[:, 1][:, None]
    dy = offs[:, 2][:, None]
    dx = offs[:, 3][:, None]
    nt = frame[None, :] + dt
    nz = z[None, :] + dz
    ny = y[None, :] + dy
    nx = x[None, :] + dx
    valid = (
        (nt >= 0) & (nt < T) & (nz >= 0) & (nz < S)
        & (ny >= 0) & (ny < S) & (nx >= 0) & (nx < S)
    )
    nlin = ((first[None, :] + dt) * S + nz) * S * S + ny * S + nx
    nlin = jnp.clip(nlin, 0, B * T * S * S * S - 1)
    idx = jnp.where(valid, grid[nlin], -1)

    feats_ext = jnp.concatenate(
        [feats, jnp.zeros((1, C), dtype=feats.dtype)], axis=0
    )
    g = feats_ext[jnp.where(idx < 0, N, idx)]

    W2 = W.reshape(NK, C, W.shape[-1])
    b2 = jnp.broadcast_to(b, (1, W.shape[-1]))

    BLKN = 2048
    R = N // BLKN
    out = pl.pallas_call(
        _matmul_body,
        grid=(R, NK),
        in_specs=[
            pl.BlockSpec((1, BLKN, C), lambda rr, kk: (kk, rr, 0)),
            pl.BlockSpec((1, C, C), lambda rr, kk: (kk, 0, 0)),
            pl.BlockSpec((1, C), lambda rr, kk: (0, 0)),
        ],
        out_specs=pl.BlockSpec((BLKN, C), lambda rr, kk: (rr, 0)),
        out_shape=jax.ShapeDtypeStruct((N, W.shape[-1]), jnp.float32),
    )(g, W2, b2)

    return out, coords


# speedup vs baseline: 2.3934x; 2.3934x over previous
import jax
import jax.numpy as jnp
import numpy as np
from jax.experimental import pallas as pl
from jax.experimental.pallas import tpu as pltpu

B = 2
T = 4
S = 32
K = 3


def _matmul_body(g_ref, w_ref, b_ref, out_ref):
    k = pl.program_id(1)

    @pl.when(k == 0)
    def _init():
        out_ref[...] = b_ref[...] + jnp.dot(
            g_ref[0], w_ref[0], preferred_element_type=jnp.float32
        )

    @pl.when(k != 0)
    def _acc():
        out_ref[...] += jnp.dot(
            g_ref[0], w_ref[0], preferred_element_type=jnp.float32
        )


def kernel(feats, coords, W, b, num_frames):
    N, C = feats.shape
    NK = K * K * K * K
    coords32 = coords.astype(jnp.int32)
    first = coords32[:, 0]
    z, y, x = coords32[:, 1], coords32[:, 2], coords32[:, 3]
    frame = first % T

    lin = ((first * S + z) * S + y) * S + x
    grid = jnp.full((B * T * S * S * S,), -1, dtype=jnp.int32)
    grid = grid.at[lin].set(jnp.arange(N, dtype=jnp.int32))

    r = K // 2
    offs = []
    for kt in range(K):
        for kz in range(K):
            for ky in range(K):
                for kx in range(K):
                    offs.append((kt - r, kz - r, ky - r, kx - r))
    offs = np.array(offs, dtype=np.int32)

    dt = offs[:, 0][:, None]
    dz = offs[:, 1][:, None]
    dy = offs[:, 2][:, None]
    dx = offs[:, 3][:, None]
    nt = frame[None, :] + dt
    nz = z[None, :] + dz
    ny = y[None, :] + dy
    nx = x[None, :] + dx
    valid = (
        (nt >= 0) & (nt < T) & (nz >= 0) & (nz < S)
        & (ny >= 0) & (ny < S) & (nx >= 0) & (nx < S)
    )
    nlin = ((first[None, :] + dt) * S + nz) * S * S + ny * S + nx
    nlin = jnp.clip(nlin, 0, B * T * S * S * S - 1)
    idx = jnp.where(valid, grid[nlin], -1)

    feats_ext = jnp.concatenate(
        [feats, jnp.zeros((1, C), dtype=feats.dtype)], axis=0
    )
    g = feats_ext[jnp.where(idx < 0, N, idx)]

    W2 = W.reshape(NK, C, W.shape[-1])
    b2 = jnp.broadcast_to(b, (1, W.shape[-1]))

    BLKN = 2048
    R = N // BLKN
    out = pl.pallas_call(
        _matmul_body,
        grid=(R, NK),
        in_specs=[
            pl.BlockSpec((1, BLKN, C), lambda rr, kk: (kk, rr, 0)),
            pl.BlockSpec((1, C, C), lambda rr, kk: (kk, 0, 0)),
            pl.BlockSpec((1, C), lambda rr, kk: (0, 0)),
        ],
        out_specs=pl.BlockSpec((BLKN, C), lambda rr, kk: (rr, 0)),
        out_shape=jax.ShapeDtypeStruct((N, W.shape[-1]), jnp.float32),
    )(g, W2, b2)

    return out, coords


# speedup vs baseline: 68.7034x; 28.7056x over previous
import functools

import jax
import jax.numpy as jnp
from jax import lax
from jax.experimental import pallas as pl
from jax.experimental.pallas import tpu as pltpu
from jax.experimental.pallas import tpu_sc as plsc

B = 2
T = 4
S = 32
N = 8192
C = 128
NK = 81

SP = S + 2
TP = T + 2
SY = SP
SZ = SP * SP
ST = SP * SP * SP
SB = TP * ST
GS_PAD = 471680

NC = 2
NS = 16
L = 16

BLK = 128
MB1 = N // BLK
CORE_OFFS = 40
CORE_MB = CORE_OFFS * MB1
MAXB = 2 * CORE_MB
TRASHB = MAXB
NBLKK = MAXB + 64
CAP = MAXB * BLK
SLOT = N + BLK
NSLOT = 3
CHUNK = 8

_mesh = plsc.VectorSubcoreMesh(
    core_axis_name="c", subcore_axis_name="s", num_cores=NC, num_subcores=NS
)


def _lane():
    return lax.iota(jnp.int32, L)


def _extract(vec, lane_idx):
    return jnp.max(jnp.where(_lane() == lane_idx, vec, jnp.int32(-2147483648)))


def _sc_build_body(lin_hbm, feats_hbm,
                   gathered_hbm, out_idx_hbm, blkk_hbm,
                   meta_hbm,
                   lin_v, nidx2d, vals2d, in_loc, out_loc, rows_v,
                   sidx_v, sval_v, idx16, val16, cnts_v, grid_sh, cnts_sh,
                   sem):
    lc = lax.axis_index("c")
    sl = lax.axis_index("s")
    lane = _lane()

    neg = jnp.full((L,), -1, jnp.int32)

    def fill_neg(i, _):
        lin_v[pl.ds(i * L, L)] = neg
        return 0
    lax.fori_loop(0, N // L, fill_neg, 0)
    mchunk = GS_PAD // NS
    mbase = sl * mchunk
    for q in range(3):
        pltpu.sync_copy(lin_v, grid_sh.at[pl.ds(mbase + q * N, N)])
    pltpu.sync_copy(
        lin_v.at[pl.ds(0, mchunk - 3 * N)],
        grid_sh.at[pl.ds(mbase + 3 * N, mchunk - 3 * N)],
    )
    pltpu.sync_copy(lin_hbm, lin_v)
    plsc.subcore_barrier()

    for ch in range(4):
        pbase = sl * 512 + ch * BLK

        def fill_scatter(m, _):
            sidx_v[pl.ds(m * L, L)] = lin_v[pl.ds(pbase + m * L, L)]
            sval_v[pl.ds(m * L, L)] = pbase + m * L + lane
            return 0
        lax.fori_loop(0, BLK // L, fill_scatter, 0)
        pltpu.sync_copy(sval_v, grid_sh.at[sidx_v])
    plsc.subcore_barrier()

    def do_offset(j):
        s_core = sl + 16 * j
        s_glob = lc * CORE_OFFS + s_core
        k_id = s_glob + (s_glob >= 40).astype(jnp.int32)
        kt = k_id // 27
        kz = (k_id // 9) % 3
        ky = (k_id // 3) % 3
        kx = k_id % 3
        delta = (kt - 1) * ST + (kz - 1) * SZ + (ky - 1) * SY + (kx - 1)

        def build_nidx(i, _):
            r = i // (BLK // L)
            m = i % (BLK // L)
            nidx2d[r, pl.ds(m * L, L)] = lin_v[pl.ds(i * L, L)] + delta
            return 0
        lax.fori_loop(0, N // L, build_nidx, 0)

        def chunk_copy(q):
            return pltpu.make_async_copy(
                grid_sh.at[nidx2d.at[q]],
                vals2d.at[q],
                sem,
            )

        for q in range(N // BLK):
            chunk_copy(q).start()
            if q >= 8:
                chunk_copy(q - 8).wait()
        for q in range(N // BLK - 8, N // BLK):
            chunk_copy(q).wait()

        lbase = j * SLOT

        def compact(i, pos):
            r = i // (BLK // L)
            mm = i % (BLK // L)
            v = vals2d[r, pl.ds(mm * L, L)]
            m = v >= 0
            mi = m.astype(jnp.int32)
            inc = plsc.cumsum(mi)
            tgt = lbase + pos + (inc - mi)
            plsc.store_scatter(in_loc, [tgt], v, mask=m)
            plsc.store_scatter(out_loc, [tgt], i * L + lane, mask=m)
            return pos + jnp.max(inc)
        cnt = lax.fori_loop(0, N // L, compact, jnp.int32(0))

        idx16[...] = jnp.where(lane == 0, s_core, 44 + lane)
        val16[...] = jnp.full((L,), 1, jnp.int32) * cnt
        pltpu.sync_copy(val16, cnts_sh.at[idx16])

    zero16 = jnp.zeros((L,), jnp.int32)
    trash16 = jnp.full((L,), N, jnp.int32)

    def prefill(i, _):
        in_loc[pl.ds(i * L, L)] = zero16
        out_loc[pl.ds(i * L, L)] = trash16
        return 0
    lax.fori_loop(0, (NSLOT * SLOT) // L, prefill, 0)

    do_offset(0)
    do_offset(1)

    @pl.when(sl < CORE_OFFS - 2 * NS)
    def _():
        do_offset(2)

    plsc.subcore_barrier()

    pltpu.sync_copy(cnts_sh.at[pl.ds(0, 48)], cnts_v)
    running = jnp.int32(0)
    ex = []
    nbv = []
    for r in range(3):
        c = cnts_v[pl.ds(r * L, L)]
        valid = (r * L + lane) < CORE_OFFS
        c = jnp.where(valid, c, 0)
        bv = (c + (BLK - 1)) >> 7
        inc = plsc.cumsum(bv)
        ex.append(inc - bv + running)
        nbv.append(bv)
        running = running + jnp.max(inc)
    nb_total = running

    @pl.when(sl == 0)
    def _():
        idx16[...] = jnp.where(lane == 0, lc, 8 + lane)
        val16[...] = jnp.full((L,), 1, jnp.int32) * nb_total
        pltpu.sync_copy(val16, meta_hbm.at[idx16])

    def emit_offset(j):
        s_core = sl + 16 * j
        s_glob = lc * CORE_OFFS + s_core
        k_id = s_glob + (s_glob >= 40).astype(jnp.int32)
        base_j = _extract(ex[j], sl)
        nb_j = _extract(nbv[j], sl)
        gb0 = lc * CORE_MB + base_j
        lbase = j * SLOT

        def do_block(b2, _):
            gb = gb0 + b2

            def stage_idx(m, _):
                sidx_v[pl.ds(m * L, L)] = in_loc[
                    pl.ds(lbase + b2 * BLK + m * L, L)
                ]
                return 0
            lax.fori_loop(0, BLK // L, stage_idx, 0)
            pltpu.sync_copy(feats_hbm.at[sidx_v], rows_v)
            pltpu.sync_copy(rows_v, gathered_hbm.at[pl.ds(gb * BLK, BLK)])
            pltpu.sync_copy(
                out_loc.at[pl.ds(lbase + b2 * BLK, BLK)],
                out_idx_hbm.at[pl.ds(gb * BLK, BLK)],
            )
            return 0
        lax.fori_loop(0, nb_j, do_block, 0)

        for v in range(4):
            @pl.when(v * L < nb_j)
            def _():
                lv = v * L + lane
                idx16[...] = jnp.where(lv < nb_j, gb0 + lv, TRASHB + lane)
                val16[...] = jnp.full((L,), 1, jnp.int32) * k_id
                pltpu.sync_copy(val16, blkk_hbm.at[idx16])

    emit_offset(0)
    emit_offset(1)

    @pl.when(sl < CORE_OFFS - 2 * NS)
    def _():
        emit_offset(2)


def _tc_body(meta_s, blkk_s, feats_r, w_r, b_r, gathered_hbm,
             outbase_r, partial_hbm, inb, outb, insem, outsem):
    outbase_r[...] = (
        jnp.dot(feats_r[...], w_r[40], preferred_element_type=jnp.float32)
        + b_r[...]
    )

    def process_region(cb0, nb):
        nch = (nb + CHUNK - 1) // CHUNK
        rows = CHUNK * BLK

        def in_copy(i, slot):
            return pltpu.make_async_copy(
                gathered_hbm.at[pl.ds((cb0 + i * CHUNK) * BLK, rows)],
                inb.at[slot],
                insem.at[slot],
            )

        def out_copy(i, slot):
            return pltpu.make_async_copy(
                outb.at[slot],
                partial_hbm.at[pl.ds((cb0 + i * CHUNK) * BLK, rows)],
                outsem.at[slot],
            )

        @pl.when(nch > 0)
        def _():
            in_copy(0, 0).start()

        def body(i, _):
            slot = lax.rem(i, 2)
            nslot = lax.rem(i + 1, 2)

            @pl.when(i + 1 < nch)
            def _():
                in_copy(i + 1, nslot).start()

            in_copy(i, slot).wait()

            @pl.when(i >= 2)
            def _():
                out_copy(i - 2, slot).wait()

            for u in range(CHUNK):
                k = blkk_s[cb0 + i * CHUNK + u]
                k = jnp.clip(k, 0, NK - 1)
                outb[slot, pl.ds(u * BLK, BLK), :] = jnp.dot(
                    inb[slot, pl.ds(u * BLK, BLK), :],
                    w_r[k],
                    preferred_element_type=jnp.float32,
                )
            out_copy(i, slot).start()
            return 0

        lax.fori_loop(0, nch, body, 0)

        @pl.when(nch >= 2)
        def _():
            out_copy(nch - 2, lax.rem(nch - 2, 2)).wait()

        @pl.when(nch >= 1)
        def _():
            out_copy(nch - 1, lax.rem(nch - 1, 2)).wait()

    process_region(0, meta_s[0])
    process_region(CORE_MB, meta_s[1])


def _sc_scatter_body(outbase_hbm, partial_hbm, out_idx_hbm, meta_hbm,
                     out_hbm, shared, idxb, rowsb, meta_v):
    lc = lax.axis_index("c")
    sl = lax.axis_index("s")

    @pl.when(lc == 0)
    def _():
        rows_per_tile = N // NS
        pltpu.sync_copy(
            outbase_hbm.at[pl.ds(sl * rows_per_tile, rows_per_tile)],
            shared.at[pl.ds(sl * rows_per_tile, rows_per_tile)],
        )
        pltpu.sync_copy(meta_hbm, meta_v)
        nb0 = _extract(meta_v[pl.ds(0, L)], jnp.int32(0))
        nb1 = _extract(meta_v[pl.ds(0, L)], jnp.int32(1))
        plsc.subcore_barrier()

        def region(cb0, nb):
            def do_block(t, _):
                gb = cb0 + sl + t * NS
                pltpu.sync_copy(out_idx_hbm.at[pl.ds(gb * BLK, BLK)], idxb)
                pltpu.sync_copy(partial_hbm.at[pl.ds(gb * BLK, BLK)], rowsb)
                pltpu.sync_copy(rowsb, shared.at[idxb], add=True)
                return 0
            ntrip = (nb - sl + NS - 1) // NS
            lax.fori_loop(0, ntrip, do_block, 0)

        region(0, nb0)
        region(CORE_MB, nb1)
        plsc.subcore_barrier()
        pltpu.sync_copy(
            shared.at[pl.ds(sl * rows_per_tile, rows_per_tile)],
            out_hbm.at[pl.ds(sl * rows_per_tile, rows_per_tile)],
        )


_sc_build = functools.partial(
    pl.kernel,
    out_type=(
        jax.ShapeDtypeStruct((CAP, C), jnp.float32),
        jax.ShapeDtypeStruct((CAP,), jnp.int32),
        jax.ShapeDtypeStruct((NBLKK,), jnp.int32),
        jax.ShapeDtypeStruct((32,), jnp.int32),
    ),
    mesh=_mesh,
    scratch_types=[
        pltpu.VMEM((N,), jnp.int32),
        pltpu.VMEM((N // BLK, BLK), jnp.int32),
        pltpu.VMEM((N // BLK, BLK), jnp.int32),
        pltpu.VMEM((NSLOT * SLOT,), jnp.int32),
        pltpu.VMEM((NSLOT * SLOT,), jnp.int32),
        pltpu.VMEM((BLK, C), jnp.float32),
        pltpu.VMEM((BLK,), jnp.int32),
        pltpu.VMEM((BLK,), jnp.int32),
        pltpu.VMEM((L,), jnp.int32),
        pltpu.VMEM((L,), jnp.int32),
        pltpu.VMEM((48,), jnp.int32),
        pltpu.VMEM_SHARED((GS_PAD,), jnp.int32),
        pltpu.VMEM_SHARED((64,), jnp.int32),
        pltpu.SemaphoreType.DMA,
    ],
    compiler_params=pltpu.CompilerParams(needs_layout_passes=False),
)(_sc_build_body)


_sc_scatter = functools.partial(
    pl.kernel,
    out_type=jax.ShapeDtypeStruct((N, C), jnp.float32),
    mesh=_mesh,
    scratch_types=[
        pltpu.VMEM_SHARED((N + 16, C), jnp.float32),
        pltpu.VMEM((BLK,), jnp.int32),
        pltpu.VMEM((BLK, C), jnp.float32),
        pltpu.VMEM((32,), jnp.int32),
    ],
    compiler_params=pltpu.CompilerParams(needs_layout_passes=False),
)(_sc_scatter_body)


def kernel(feats, coords, W, b, num_frames):
    coords32 = coords.astype(jnp.int32)
    first = coords32[:, 0]
    z, y, x = coords32[:, 1], coords32[:, 2], coords32[:, 3]
    b_idx = first // T
    t = first % T
    lin_pad = b_idx * SB + (t + 1) * ST + (z + 1) * SZ + (y + 1) * SY + (x + 1)

    gathered, out_idx, blkk, meta = _sc_build(lin_pad, feats)

    W2 = W.reshape(NK, C, C)
    b2 = b.reshape(1, C)
    outbase, partial = pl.pallas_call(
        _tc_body,
        in_specs=[
            pl.BlockSpec(memory_space=pltpu.SMEM),
            pl.BlockSpec(memory_space=pltpu.SMEM),
            pl.BlockSpec(memory_space=pltpu.VMEM),
            pl.BlockSpec(memory_space=pltpu.VMEM),
            pl.BlockSpec(memory_space=pltpu.VMEM),
            pl.BlockSpec(memory_space=pl.ANY),
        ],
        out_specs=[
            pl.BlockSpec(memory_space=pltpu.VMEM),
            pl.BlockSpec(memory_space=pl.ANY),
        ],
        out_shape=[
            jax.ShapeDtypeStruct((N, C), jnp.float32),
            jax.ShapeDtypeStruct((CAP, C), jnp.float32),
        ],
        scratch_shapes=[
            pltpu.VMEM((2, CHUNK * BLK, C), jnp.float32),
            pltpu.VMEM((2, CHUNK * BLK, C), jnp.float32),
            pltpu.SemaphoreType.DMA((2,)),
            pltpu.SemaphoreType.DMA((2,)),
        ],
    )(meta, blkk, feats, W2, b2, gathered)

    out = _sc_scatter(outbase, partial, out_idx, meta)
    return out, coords


# speedup vs baseline: 69.2763x; 1.0083x over previous
import functools

import jax
import jax.numpy as jnp
from jax import lax
from jax.experimental import pallas as pl
from jax.experimental.pallas import tpu as pltpu
from jax.experimental.pallas import tpu_sc as plsc

B = 2
T = 4
S = 32
N = 8192
C = 128
NK = 81

SP = S + 2
TP = T + 2
SY = SP
SZ = SP * SP
ST = SP * SP * SP
SB = TP * ST
GS_PAD = 471680

NC = 2
NS = 16
L = 16

BLK = 128
MB1 = N // BLK
CORE_OFFS = 40
CORE_MB = CORE_OFFS * MB1
MAXB = 2 * CORE_MB
TRASHB = MAXB
NBLKK = MAXB + 64
CAP = MAXB * BLK
SLOT = N + BLK
NSLOT = 3
CHUNK = 8

_mesh = plsc.VectorSubcoreMesh(
    core_axis_name="c", subcore_axis_name="s", num_cores=NC, num_subcores=NS
)


def _lane():
    return lax.iota(jnp.int32, L)


def _extract(vec, lane_idx):
    return jnp.max(jnp.where(_lane() == lane_idx, vec, jnp.int32(-2147483648)))


def _sc_build_body(lin_hbm, feats_hbm,
                   gathered_hbm, out_idx_hbm, blkk_hbm,
                   meta_hbm,
                   lin_v, nidx2d, vals2d, in_loc, out_loc, rows_v,
                   sidx_v, sval_v, idx16, val16, cnts_v, grid_sh, cnts_sh,
                   sem):
    lc = lax.axis_index("c")
    sl = lax.axis_index("s")
    lane = _lane()

    neg = jnp.full((L,), -1, jnp.int32)

    def fill_neg(i, _):
        lin_v[pl.ds(i * L, L)] = neg
        return 0
    lax.fori_loop(0, N // L, fill_neg, 0)
    mchunk = GS_PAD // NS
    mbase = sl * mchunk
    for q in range(3):
        pltpu.sync_copy(lin_v, grid_sh.at[pl.ds(mbase + q * N, N)])
    pltpu.sync_copy(
        lin_v.at[pl.ds(0, mchunk - 3 * N)],
        grid_sh.at[pl.ds(mbase + 3 * N, mchunk - 3 * N)],
    )
    pltpu.sync_copy(lin_hbm, lin_v)
    plsc.subcore_barrier()

    for ch in range(4):
        pbase = sl * 512 + ch * BLK

        def fill_scatter(m, _):
            sidx_v[pl.ds(m * L, L)] = lin_v[pl.ds(pbase + m * L, L)]
            sval_v[pl.ds(m * L, L)] = pbase + m * L + lane
            return 0
        lax.fori_loop(0, BLK // L, fill_scatter, 0)
        pltpu.sync_copy(sval_v, grid_sh.at[sidx_v])
    plsc.subcore_barrier()

    def do_offset(j):
        s_core = sl + 16 * j
        s_glob = lc * CORE_OFFS + s_core
        k_id = s_glob + (s_glob >= 40).astype(jnp.int32)
        kt = k_id // 27
        kz = (k_id // 9) % 3
        ky = (k_id // 3) % 3
        kx = k_id % 3
        delta = (kt - 1) * ST + (kz - 1) * SZ + (ky - 1) * SY + (kx - 1)

        def build_nidx(i, _):
            r = i // (BLK // L)
            m = i % (BLK // L)
            nidx2d[r, pl.ds(m * L, L)] = lin_v[pl.ds(i * L, L)] + delta
            return 0
        lax.fori_loop(0, N // L, build_nidx, 0)

        def chunk_copy(q):
            return pltpu.make_async_copy(
                grid_sh.at[nidx2d.at[q]],
                vals2d.at[q],
                sem,
            )

        W_DEPTH = 16
        for q in range(N // BLK):
            chunk_copy(q).start()
            if q >= W_DEPTH:
                chunk_copy(q - W_DEPTH).wait()
        for q in range(N // BLK - W_DEPTH, N // BLK):
            chunk_copy(q).wait()

        lbase = j * SLOT

        def compact(i, pos_v):
            r = i // (BLK // L)
            mm = i % (BLK // L)
            v = vals2d[r, pl.ds(mm * L, L)]
            m = v >= 0
            mi = m.astype(jnp.int32)
            inc = plsc.cumsum(mi)
            tgt = lbase + pos_v + (inc - mi)
            plsc.store_scatter(in_loc, [tgt], v, mask=m)
            plsc.store_scatter(out_loc, [tgt], i * L + lane, mask=m)
            return pos_v + plsc.all_reduce_population_count(m)
        pos_v = lax.fori_loop(0, N // L, compact,
                              jnp.zeros((L,), jnp.int32))
        cnt = jnp.max(pos_v)

        idx16[...] = jnp.where(lane == 0, s_core, 44 + lane)
        val16[...] = jnp.full((L,), 1, jnp.int32) * cnt
        pltpu.sync_copy(val16, cnts_sh.at[idx16])

    zero16 = jnp.zeros((L,), jnp.int32)
    trash16 = jnp.full((L,), N, jnp.int32)

    def prefill(i, _):
        in_loc[pl.ds(i * L, L)] = zero16
        out_loc[pl.ds(i * L, L)] = trash16
        return 0
    lax.fori_loop(0, (NSLOT * SLOT) // L, prefill, 0)

    do_offset(0)
    do_offset(1)

    @pl.when(sl < CORE_OFFS - 2 * NS)
    def _():
        do_offset(2)

    plsc.subcore_barrier()

    pltpu.sync_copy(cnts_sh.at[pl.ds(0, 48)], cnts_v)
    running = jnp.int32(0)
    ex = []
    nbv = []
    for r in range(3):
        c = cnts_v[pl.ds(r * L, L)]
        valid = (r * L + lane) < CORE_OFFS
        c = jnp.where(valid, c, 0)
        bv = (c + (BLK - 1)) >> 7
        inc = plsc.cumsum(bv)
        ex.append(inc - bv + running)
        nbv.append(bv)
        running = running + jnp.max(inc)
    nb_total = running

    @pl.when(sl == 0)
    def _():
        idx16[...] = jnp.where(lane == 0, lc, 8 + lane)
        val16[...] = jnp.full((L,), 1, jnp.int32) * nb_total
        pltpu.sync_copy(val16, meta_hbm.at[idx16])

    def emit_offset(j):
        s_core = sl + 16 * j
        s_glob = lc * CORE_OFFS + s_core
        k_id = s_glob + (s_glob >= 40).astype(jnp.int32)
        base_j = _extract(ex[j], sl)
        nb_j = _extract(nbv[j], sl)
        gb0 = lc * CORE_MB + base_j
        lbase = j * SLOT

        def do_block(b2, _):
            gb = gb0 + b2

            def stage_idx(m, _):
                sidx_v[pl.ds(m * L, L)] = in_loc[
                    pl.ds(lbase + b2 * BLK + m * L, L)
                ]
                return 0
            lax.fori_loop(0, BLK // L, stage_idx, 0)
            pltpu.sync_copy(feats_hbm.at[sidx_v], rows_v)
            pltpu.sync_copy(rows_v, gathered_hbm.at[pl.ds(gb * BLK, BLK)])
            pltpu.sync_copy(
                out_loc.at[pl.ds(lbase + b2 * BLK, BLK)],
                out_idx_hbm.at[pl.ds(gb * BLK, BLK)],
            )
            return 0
        lax.fori_loop(0, nb_j, do_block, 0)

        for v in range(4):
            @pl.when(v * L < nb_j)
            def _():
                lv = v * L + lane
                idx16[...] = jnp.where(lv < nb_j, gb0 + lv, TRASHB + lane)
                val16[...] = jnp.full((L,), 1, jnp.int32) * k_id
                pltpu.sync_copy(val16, blkk_hbm.at[idx16])

    emit_offset(0)
    emit_offset(1)

    @pl.when(sl < CORE_OFFS - 2 * NS)
    def _():
        emit_offset(2)


def _tc_body(meta_s, blkk_s, feats_r, w_r, b_r, gathered_hbm,
             outbase_r, partial_hbm, inb, outb, insem, outsem):
    outbase_r[...] = (
        jnp.dot(feats_r[...], w_r[40], preferred_element_type=jnp.float32)
        + b_r[...]
    )

    def process_region(cb0, nb):
        nch = (nb + CHUNK - 1) // CHUNK
        rows = CHUNK * BLK

        def in_copy(i, slot):
            return pltpu.make_async_copy(
                gathered_hbm.at[pl.ds((cb0 + i * CHUNK) * BLK, rows)],
                inb.at[slot],
                insem.at[slot],
            )

        def out_copy(i, slot):
            return pltpu.make_async_copy(
                outb.at[slot],
                partial_hbm.at[pl.ds((cb0 + i * CHUNK) * BLK, rows)],
                outsem.at[slot],
            )

        @pl.when(nch > 0)
        def _():
            in_copy(0, 0).start()

        def body(i, _):
            slot = lax.rem(i, 2)
            nslot = lax.rem(i + 1, 2)

            @pl.when(i + 1 < nch)
            def _():
                in_copy(i + 1, nslot).start()

            in_copy(i, slot).wait()

            @pl.when(i >= 2)
            def _():
                out_copy(i - 2, slot).wait()

            for u in range(CHUNK):
                k = blkk_s[cb0 + i * CHUNK + u]
                k = jnp.clip(k, 0, NK - 1)
                outb[slot, pl.ds(u * BLK, BLK), :] = jnp.dot(
                    inb[slot, pl.ds(u * BLK, BLK), :],
                    w_r[k],
                    preferred_element_type=jnp.float32,
                )
            out_copy(i, slot).start()
            return 0

        lax.fori_loop(0, nch, body, 0)

        @pl.when(nch >= 2)
        def _():
            out_copy(nch - 2, lax.rem(nch - 2, 2)).wait()

        @pl.when(nch >= 1)
        def _():
            out_copy(nch - 1, lax.rem(nch - 1, 2)).wait()

    process_region(0, meta_s[0])
    process_region(CORE_MB, meta_s[1])


def _sc_scatter_body(outbase_hbm, partial_hbm, out_idx_hbm, meta_hbm,
                     out_hbm, shared, idxb, rowsb, meta_v):
    lc = lax.axis_index("c")
    sl = lax.axis_index("s")

    @pl.when(lc == 0)
    def _():
        rows_per_tile = N // NS
        pltpu.sync_copy(
            outbase_hbm.at[pl.ds(sl * rows_per_tile, rows_per_tile)],
            shared.at[pl.ds(sl * rows_per_tile, rows_per_tile)],
        )
        pltpu.sync_copy(meta_hbm, meta_v)
        nb0 = _extract(meta_v[pl.ds(0, L)], jnp.int32(0))
        nb1 = _extract(meta_v[pl.ds(0, L)], jnp.int32(1))
        plsc.subcore_barrier()

        def region(cb0, nb):
            def do_block(t, _):
                gb = cb0 + sl + t * NS
                pltpu.sync_copy(out_idx_hbm.at[pl.ds(gb * BLK, BLK)], idxb)
                pltpu.sync_copy(partial_hbm.at[pl.ds(gb * BLK, BLK)], rowsb)
                pltpu.sync_copy(rowsb, shared.at[idxb], add=True)
                return 0
            ntrip = (nb - sl + NS - 1) // NS
            lax.fori_loop(0, ntrip, do_block, 0)

        region(0, nb0)
        region(CORE_MB, nb1)
        plsc.subcore_barrier()
        pltpu.sync_copy(
            shared.at[pl.ds(sl * rows_per_tile, rows_per_tile)],
            out_hbm.at[pl.ds(sl * rows_per_tile, rows_per_tile)],
        )


_sc_build = functools.partial(
    pl.kernel,
    out_type=(
        jax.ShapeDtypeStruct((CAP, C), jnp.float32),
        jax.ShapeDtypeStruct((CAP,), jnp.int32),
        jax.ShapeDtypeStruct((NBLKK,), jnp.int32),
        jax.ShapeDtypeStruct((32,), jnp.int32),
    ),
    mesh=_mesh,
    scratch_types=[
        pltpu.VMEM((N,), jnp.int32),
        pltpu.VMEM((N // BLK, BLK), jnp.int32),
        pltpu.VMEM((N // BLK, BLK), jnp.int32),
        pltpu.VMEM((NSLOT * SLOT,), jnp.int32),
        pltpu.VMEM((NSLOT * SLOT,), jnp.int32),
        pltpu.VMEM((BLK, C), jnp.float32),
        pltpu.VMEM((BLK,), jnp.int32),
        pltpu.VMEM((BLK,), jnp.int32),
        pltpu.VMEM((L,), jnp.int32),
        pltpu.VMEM((L,), jnp.int32),
        pltpu.VMEM((48,), jnp.int32),
        pltpu.VMEM_SHARED((GS_PAD,), jnp.int32),
        pltpu.VMEM_SHARED((64,), jnp.int32),
        pltpu.SemaphoreType.DMA,
    ],
    compiler_params=pltpu.CompilerParams(needs_layout_passes=False),
)(_sc_build_body)


_sc_scatter = functools.partial(
    pl.kernel,
    out_type=jax.ShapeDtypeStruct((N, C), jnp.float32),
    mesh=_mesh,
    scratch_types=[
        pltpu.VMEM_SHARED((N + 16, C), jnp.float32),
        pltpu.VMEM((BLK,), jnp.int32),
        pltpu.VMEM((BLK, C), jnp.float32),
        pltpu.VMEM((32,), jnp.int32),
    ],
    compiler_params=pltpu.CompilerParams(needs_layout_passes=False),
)(_sc_scatter_body)


def kernel(feats, coords, W, b, num_frames):
    coords32 = coords.astype(jnp.int32)
    first = coords32[:, 0]
    z, y, x = coords32[:, 1], coords32[:, 2], coords32[:, 3]
    b_idx = first // T
    t = first % T
    lin_pad = b_idx * SB + (t + 1) * ST + (z + 1) * SZ + (y + 1) * SY + (x + 1)

    gathered, out_idx, blkk, meta = _sc_build(lin_pad, feats)

    W2 = W.reshape(NK, C, C)
    b2 = b.reshape(1, C)
    outbase, partial = pl.pallas_call(
        _tc_body,
        in_specs=[
            pl.BlockSpec(memory_space=pltpu.SMEM),
            pl.BlockSpec(memory_space=pltpu.SMEM),
            pl.BlockSpec(memory_space=pltpu.VMEM),
            pl.BlockSpec(memory_space=pltpu.VMEM),
            pl.BlockSpec(memory_space=pltpu.VMEM),
            pl.BlockSpec(memory_space=pl.ANY),
        ],
        out_specs=[
            pl.BlockSpec(memory_space=pltpu.VMEM),
            pl.BlockSpec(memory_space=pl.ANY),
        ],
        out_shape=[
            jax.ShapeDtypeStruct((N, C), jnp.float32),
            jax.ShapeDtypeStruct((CAP, C), jnp.float32),
        ],
        scratch_shapes=[
            pltpu.VMEM((2, CHUNK * BLK, C), jnp.float32),
            pltpu.VMEM((2, CHUNK * BLK, C), jnp.float32),
            pltpu.SemaphoreType.DMA((2,)),
            pltpu.SemaphoreType.DMA((2,)),
        ],
    )(meta, blkk, feats, W2, b2, gathered)

    out = _sc_scatter(outbase, partial, out_idx, meta)
    return out, coords


# speedup vs baseline: 69.3519x; 1.0011x over previous
import functools

import jax
import jax.numpy as jnp
from jax import lax
from jax.experimental import pallas as pl
from jax.experimental.pallas import tpu as pltpu
from jax.experimental.pallas import tpu_sc as plsc

B = 2
T = 4
S = 32
N = 8192
C = 128
NK = 81

SP = S + 2
TP = T + 2
SY = SP
SZ = SP * SP
ST = SP * SP * SP
SB = TP * ST
GS_PAD = 471680

NC = 2
NS = 16
L = 16

BLK = 128
MB1 = N // BLK
CORE_OFFS = 40
CORE_MB = CORE_OFFS * MB1
MAXB = 2 * CORE_MB
TRASHB = MAXB
NBLKK = MAXB + 64
CAP = MAXB * BLK
SLOT = N + BLK
NSLOT = 3
CHUNK = 8

_mesh = plsc.VectorSubcoreMesh(
    core_axis_name="c", subcore_axis_name="s", num_cores=NC, num_subcores=NS
)


def _lane():
    return lax.iota(jnp.int32, L)


def _extract(vec, lane_idx):
    return jnp.max(jnp.where(_lane() == lane_idx, vec, jnp.int32(-2147483648)))


def _sc_build_body(lin_hbm, feats_hbm,
                   gathered_hbm, out_idx_hbm, blkk_hbm,
                   meta_hbm,
                   lin_v, nidx2d, vals2d, in_loc, out_loc, rows_v,
                   sidx_v, sval_v, idx16, val16, cnts_v, grid_sh, cnts_sh,
                   sem):
    lc = lax.axis_index("c")
    sl = lax.axis_index("s")
    lane = _lane()

    neg = jnp.full((L,), -1, jnp.int32)

    def fill_neg(i, _):
        lin_v[pl.ds(i * L, L)] = neg
        return 0
    lax.fori_loop(0, N // L, fill_neg, 0)
    mchunk = GS_PAD // NS
    mbase = sl * mchunk
    mcopies = [
        pltpu.make_async_copy(
            lin_v, grid_sh.at[pl.ds(mbase + q * N, N)], sem
        )
        for q in range(3)
    ] + [
        pltpu.make_async_copy(
            lin_v.at[pl.ds(0, mchunk - 3 * N)],
            grid_sh.at[pl.ds(mbase + 3 * N, mchunk - 3 * N)],
            sem,
        )
    ]
    for mc in mcopies:
        mc.start()
    for mc in mcopies:
        mc.wait()
    pltpu.sync_copy(lin_hbm, lin_v)
    plsc.subcore_barrier()

    for ch in range(4):
        pbase = sl * 512 + ch * BLK

        def fill_scatter(m, _):
            sidx_v[ch, pl.ds(m * L, L)] = lin_v[pl.ds(pbase + m * L, L)]
            sval_v[ch, pl.ds(m * L, L)] = pbase + m * L + lane
            return 0
        lax.fori_loop(0, BLK // L, fill_scatter, 0)
        pltpu.make_async_copy(
            sval_v.at[ch], grid_sh.at[sidx_v.at[ch]], sem
        ).start()
    for ch in range(4):
        pltpu.make_async_copy(
            sval_v.at[ch], grid_sh.at[sidx_v.at[ch]], sem
        ).wait()
    plsc.subcore_barrier()

    def do_offset(j):
        s_core = sl + 16 * j
        s_glob = lc * CORE_OFFS + s_core
        k_id = s_glob + (s_glob >= 40).astype(jnp.int32)
        kt = k_id // 27
        kz = (k_id // 9) % 3
        ky = (k_id // 3) % 3
        kx = k_id % 3
        delta = (kt - 1) * ST + (kz - 1) * SZ + (ky - 1) * SY + (kx - 1)

        def build_nidx(i, _):
            r = i // (BLK // L)
            m = i % (BLK // L)
            nidx2d[r, pl.ds(m * L, L)] = lin_v[pl.ds(i * L, L)] + delta
            return 0
        lax.fori_loop(0, N // L, build_nidx, 0)

        def chunk_copy(q):
            return pltpu.make_async_copy(
                grid_sh.at[nidx2d.at[q]],
                vals2d.at[q],
                sem,
            )

        W_DEPTH = 16
        for q in range(N // BLK):
            chunk_copy(q).start()
            if q >= W_DEPTH:
                chunk_copy(q - W_DEPTH).wait()
        for q in range(N // BLK - W_DEPTH, N // BLK):
            chunk_copy(q).wait()

        lbase = j * SLOT

        def compact(i, pos_v):
            r = i // (BLK // L)
            mm = i % (BLK // L)
            v = vals2d[r, pl.ds(mm * L, L)]
            m = v >= 0
            mi = m.astype(jnp.int32)
            inc = plsc.cumsum(mi)
            tgt = lbase + pos_v + (inc - mi)
            plsc.store_scatter(in_loc, [tgt], v, mask=m)
            plsc.store_scatter(out_loc, [tgt], i * L + lane, mask=m)
            return pos_v + plsc.all_reduce_population_count(m)
        pos_v = lax.fori_loop(0, N // L, compact,
                              jnp.zeros((L,), jnp.int32))
        cnt = jnp.max(pos_v)

        idx16[...] = jnp.where(lane == 0, s_core, 44 + lane)
        val16[...] = jnp.full((L,), 1, jnp.int32) * cnt
        pltpu.sync_copy(val16, cnts_sh.at[idx16])

    zero16 = jnp.zeros((L,), jnp.int32)
    trash16 = jnp.full((L,), N, jnp.int32)

    def prefill(i, _):
        in_loc[pl.ds(i * L, L)] = zero16
        out_loc[pl.ds(i * L, L)] = trash16
        return 0
    lax.fori_loop(0, (NSLOT * SLOT) // L, prefill, 0)

    do_offset(0)
    do_offset(1)

    @pl.when(sl < CORE_OFFS - 2 * NS)
    def _():
        do_offset(2)

    plsc.subcore_barrier()

    pltpu.sync_copy(cnts_sh.at[pl.ds(0, 48)], cnts_v)
    running = jnp.int32(0)
    ex = []
    nbv = []
    for r in range(3):
        c = cnts_v[pl.ds(r * L, L)]
        valid = (r * L + lane) < CORE_OFFS
        c = jnp.where(valid, c, 0)
        bv = (c + (BLK - 1)) >> 7
        inc = plsc.cumsum(bv)
        ex.append(inc - bv + running)
        nbv.append(bv)
        running = running + jnp.max(inc)
    nb_total = running

    @pl.when(sl == 0)
    def _():
        idx16[...] = jnp.where(lane == 0, lc, 8 + lane)
        val16[...] = jnp.full((L,), 1, jnp.int32) * nb_total
        pltpu.sync_copy(val16, meta_hbm.at[idx16])

    def emit_offset(j):
        s_core = sl + 16 * j
        s_glob = lc * CORE_OFFS + s_core
        k_id = s_glob + (s_glob >= 40).astype(jnp.int32)
        base_j = _extract(ex[j], sl)
        nb_j = _extract(nbv[j], sl)
        gb0 = lc * CORE_MB + base_j
        lbase = j * SLOT

        def do_block(b2, _):
            gb = gb0 + b2

            def stage_idx(m, _):
                sidx_v[0, pl.ds(m * L, L)] = in_loc[
                    pl.ds(lbase + b2 * BLK + m * L, L)
                ]
                return 0
            lax.fori_loop(0, BLK // L, stage_idx, 0)
            pltpu.sync_copy(feats_hbm.at[sidx_v.at[0]], rows_v)
            pltpu.sync_copy(rows_v, gathered_hbm.at[pl.ds(gb * BLK, BLK)])
            pltpu.sync_copy(
                out_loc.at[pl.ds(lbase + b2 * BLK, BLK)],
                out_idx_hbm.at[pl.ds(gb * BLK, BLK)],
            )
            return 0
        lax.fori_loop(0, nb_j, do_block, 0)

        for v in range(4):
            @pl.when(v * L < nb_j)
            def _():
                lv = v * L + lane
                idx16[...] = jnp.where(lv < nb_j, gb0 + lv, TRASHB + lane)
                val16[...] = jnp.full((L,), 1, jnp.int32) * k_id
                pltpu.sync_copy(val16, blkk_hbm.at[idx16])

    emit_offset(0)
    emit_offset(1)

    @pl.when(sl < CORE_OFFS - 2 * NS)
    def _():
        emit_offset(2)


def _tc_body(meta_s, blkk_s, feats_r, w_r, b_r, gathered_hbm,
             outbase_r, partial_hbm, inb, outb, insem, outsem):
    outbase_r[...] = (
        jnp.dot(feats_r[...], w_r[40], preferred_element_type=jnp.float32)
        + b_r[...]
    )

    def process_region(cb0, nb):
        nch = (nb + CHUNK - 1) // CHUNK
        rows = CHUNK * BLK

        def in_copy(i, slot):
            return pltpu.make_async_copy(
                gathered_hbm.at[pl.ds((cb0 + i * CHUNK) * BLK, rows)],
                inb.at[slot],
                insem.at[slot],
            )

        def out_copy(i, slot):
            return pltpu.make_async_copy(
                outb.at[slot],
                partial_hbm.at[pl.ds((cb0 + i * CHUNK) * BLK, rows)],
                outsem.at[slot],
            )

        @pl.when(nch > 0)
        def _():
            in_copy(0, 0).start()

        def body(i, _):
            slot = lax.rem(i, 2)
            nslot = lax.rem(i + 1, 2)

            @pl.when(i + 1 < nch)
            def _():
                in_copy(i + 1, nslot).start()

            in_copy(i, slot).wait()

            @pl.when(i >= 2)
            def _():
                out_copy(i - 2, slot).wait()

            for u in range(CHUNK):
                k = blkk_s[cb0 + i * CHUNK + u]
                k = jnp.clip(k, 0, NK - 1)
                outb[slot, pl.ds(u * BLK, BLK), :] = jnp.dot(
                    inb[slot, pl.ds(u * BLK, BLK), :],
                    w_r[k],
                    preferred_element_type=jnp.float32,
                )
            out_copy(i, slot).start()
            return 0

        lax.fori_loop(0, nch, body, 0)

        @pl.when(nch >= 2)
        def _():
            out_copy(nch - 2, lax.rem(nch - 2, 2)).wait()

        @pl.when(nch >= 1)
        def _():
            out_copy(nch - 1, lax.rem(nch - 1, 2)).wait()

    process_region(0, meta_s[0])
    process_region(CORE_MB, meta_s[1])


def _sc_scatter_body(outbase_hbm, partial_hbm, out_idx_hbm, meta_hbm,
                     out_hbm, shared, idxb, rowsb, meta_v):
    lc = lax.axis_index("c")
    sl = lax.axis_index("s")

    @pl.when(lc == 0)
    def _():
        rows_per_tile = N // NS
        pltpu.sync_copy(
            outbase_hbm.at[pl.ds(sl * rows_per_tile, rows_per_tile)],
            shared.at[pl.ds(sl * rows_per_tile, rows_per_tile)],
        )
        pltpu.sync_copy(meta_hbm, meta_v)
        nb0 = _extract(meta_v[pl.ds(0, L)], jnp.int32(0))
        nb1 = _extract(meta_v[pl.ds(0, L)], jnp.int32(1))
        plsc.subcore_barrier()

        def region(cb0, nb):
            def do_block(t, _):
                gb = cb0 + sl + t * NS
                pltpu.sync_copy(out_idx_hbm.at[pl.ds(gb * BLK, BLK)], idxb)
                pltpu.sync_copy(partial_hbm.at[pl.ds(gb * BLK, BLK)], rowsb)
                pltpu.sync_copy(rowsb, shared.at[idxb], add=True)
                return 0
            ntrip = (nb - sl + NS - 1) // NS
            lax.fori_loop(0, ntrip, do_block, 0)

        region(0, nb0)
        region(CORE_MB, nb1)
        plsc.subcore_barrier()
        pltpu.sync_copy(
            shared.at[pl.ds(sl * rows_per_tile, rows_per_tile)],
            out_hbm.at[pl.ds(sl * rows_per_tile, rows_per_tile)],
        )


_sc_build = functools.partial(
    pl.kernel,
    out_type=(
        jax.ShapeDtypeStruct((CAP, C), jnp.float32),
        jax.ShapeDtypeStruct((CAP,), jnp.int32),
        jax.ShapeDtypeStruct((NBLKK,), jnp.int32),
        jax.ShapeDtypeStruct((32,), jnp.int32),
    ),
    mesh=_mesh,
    scratch_types=[
        pltpu.VMEM((N,), jnp.int32),
        pltpu.VMEM((N // BLK, BLK), jnp.int32),
        pltpu.VMEM((N // BLK, BLK), jnp.int32),
        pltpu.VMEM((NSLOT * SLOT,), jnp.int32),
        pltpu.VMEM((NSLOT * SLOT,), jnp.int32),
        pltpu.VMEM((BLK, C), jnp.float32),
        pltpu.VMEM((4, BLK), jnp.int32),
        pltpu.VMEM((4, BLK), jnp.int32),
        pltpu.VMEM((L,), jnp.int32),
        pltpu.VMEM((L,), jnp.int32),
        pltpu.VMEM((48,), jnp.int32),
        pltpu.VMEM_SHARED((GS_PAD,), jnp.int32),
        pltpu.VMEM_SHARED((64,), jnp.int32),
        pltpu.SemaphoreType.DMA,
    ],
    compiler_params=pltpu.CompilerParams(needs_layout_passes=False),
)(_sc_build_body)


_sc_scatter = functools.partial(
    pl.kernel,
    out_type=jax.ShapeDtypeStruct((N, C), jnp.float32),
    mesh=_mesh,
    scratch_types=[
        pltpu.VMEM_SHARED((N + 16, C), jnp.float32),
        pltpu.VMEM((BLK,), jnp.int32),
        pltpu.VMEM((BLK, C), jnp.float32),
        pltpu.VMEM((32,), jnp.int32),
    ],
    compiler_params=pltpu.CompilerParams(needs_layout_passes=False),
)(_sc_scatter_body)


def kernel(feats, coords, W, b, num_frames):
    coords32 = coords.astype(jnp.int32)
    first = coords32[:, 0]
    z, y, x = coords32[:, 1], coords32[:, 2], coords32[:, 3]
    b_idx = first // T
    t = first % T
    lin_pad = b_idx * SB + (t + 1) * ST + (z + 1) * SZ + (y + 1) * SY + (x + 1)

    gathered, out_idx, blkk, meta = _sc_build(lin_pad, feats)

    W2 = W.reshape(NK, C, C)
    b2 = b.reshape(1, C)
    outbase, partial = pl.pallas_call(
        _tc_body,
        in_specs=[
            pl.BlockSpec(memory_space=pltpu.SMEM),
            pl.BlockSpec(memory_space=pltpu.SMEM),
            pl.BlockSpec(memory_space=pltpu.VMEM),
            pl.BlockSpec(memory_space=pltpu.VMEM),
            pl.BlockSpec(memory_space=pltpu.VMEM),
            pl.BlockSpec(memory_space=pl.ANY),
        ],
        out_specs=[
            pl.BlockSpec(memory_space=pltpu.VMEM),
            pl.BlockSpec(memory_space=pl.ANY),
        ],
        out_shape=[
            jax.ShapeDtypeStruct((N, C), jnp.float32),
            jax.ShapeDtypeStruct((CAP, C), jnp.float32),
        ],
        scratch_shapes=[
            pltpu.VMEM((2, CHUNK * BLK, C), jnp.float32),
            pltpu.VMEM((2, CHUNK * BLK, C), jnp.float32),
            pltpu.SemaphoreType.DMA((2,)),
            pltpu.SemaphoreType.DMA((2,)),
        ],
    )(meta, blkk, feats, W2, b2, gathered)

    out = _sc_scatter(outbase, partial, out_idx, meta)
    return out, coords
